# baseline (speedup 1.0000x reference)
import functools

import jax
import jax.numpy as jnp
from jax import lax
from jax.experimental import pallas as pl
from jax.experimental.pallas import tpu as pltpu
from jax.experimental.pallas import tpu_sc as plsc

_EMB_D = 768
_KEY_D = 768
_POOL = 1024
_P_LEN = 8
_TOP_K = 8
_B = 1024
_HALF = _P_LEN // 2
_NROWS = _B * _TOP_K
_L = 128
_SLAB = _HALF * _EMB_D // _L

_BB = 256
_NEG = jnp.finfo(jnp.float32).min

_NC = 2
_NS = 16
_NW = _NC * _NS
_G = _TOP_K
_NCH = _NROWS // _NW // _G
_NB = 4


def _topk_body(q_ref, k_ref, idx_ref, idxv_ref, loss_ref):
    i = pl.program_id(0)
    k = k_ref[...]
    kn = k / jnp.maximum(jnp.sqrt(jnp.sum(k * k, axis=1, keepdims=True)), 1e-12)
    q = q_ref[...]
    qn = q / jnp.maximum(jnp.sqrt(jnp.sum(q * q, axis=1, keepdims=True)), 1e-12)
    cos = lax.dot_general(qn, kn, (((1,), (1,)), ((), ())),
                          preferred_element_type=jnp.float32)
    col = lax.broadcasted_iota(jnp.int32, cos.shape, 1)
    s = cos
    idx_cols = []
    loss_acc = jnp.float32(0.0)
    for _ in range(_TOP_K):
        m = jnp.max(s, axis=1, keepdims=True)
        am = jnp.min(jnp.where(s == m, col, _POOL), axis=1,
                     keepdims=True)
        idx_cols.append(am)
        loss_acc = loss_acc + jnp.sum(1.0 - m)
        s = jnp.where(col == am, _NEG, s)
    idx = jnp.concatenate(idx_cols, axis=1)
    idx_ref[...] = idx
    idxv_ref[...] = idx * 2 + 1

    @pl.when(i == 0)
    def _():
        loss_ref[0, 0] = 0.0

    loss_ref[0, 0] += loss_acc / _B


def _topk_call(query, k_0):
    grid = _B // _BB
    return pl.pallas_call(
        _topk_body,
        grid=(grid,),
        in_specs=[
            pl.BlockSpec((_BB, _KEY_D), lambda i: (i, 0)),
            pl.BlockSpec((_POOL, _KEY_D), lambda i: (0, 0)),
        ],
        out_specs=[
            pl.BlockSpec((_BB, _TOP_K), lambda i: (i, 0)),
            pl.BlockSpec((_BB, _TOP_K), lambda i: (i, 0)),
            pl.BlockSpec((1, 1), lambda i: (0, 0),
                         memory_space=pltpu.SMEM),
        ],
        out_shape=[
            jax.ShapeDtypeStruct((_B, _TOP_K), jnp.int32),
            jax.ShapeDtypeStruct((_B, _TOP_K), jnp.int32),
            jax.ShapeDtypeStruct((1, 1), jnp.float32),
        ],
    )(query, k_0)


def _kgather_body(idx_ref, *refs):
    ins = refs[:_TOP_K]
    out_ref = refs[_TOP_K]
    for j in range(_TOP_K):
        out_ref[pl.ds(_SLAB * j, _SLAB), :] = ins[j][...]


def _kgather_call(p3, idx_flat):
    in_specs = [
        pl.BlockSpec((_SLAB, _L),
                     (lambda b, idx_ref, j=j: (idx_ref[b * _TOP_K + j] * 2, 0)))
        for j in range(_TOP_K)
    ]
    return pl.pallas_call(
        _kgather_body,
        grid_spec=pltpu.PrefetchScalarGridSpec(
            num_scalar_prefetch=1,
            grid=(_B,),
            in_specs=in_specs,
            out_specs=pl.BlockSpec((_TOP_K * _SLAB, _L), lambda b, idx_ref: (b, 0)),
        ),
        out_shape=jax.ShapeDtypeStruct((_B * _TOP_K * _SLAB, _L), jnp.float32),
    )(idx_flat, *([p3] * _TOP_K))


def _vgather_body(table, idx, outv, idx_s,
                  buf0, buf1, buf2, buf3,
                  gs0, gs1, gs2, gs3, ws0, ws1, ws2, ws3):
    c = lax.axis_index("c")
    s = lax.axis_index("s")
    wid = s * _NC + c
    base_b = wid * _NCH
    pltpu.sync_copy(idx.at[wid], idx_s)

    bufs = (buf0, buf1, buf2, buf3)
    gsems = (gs0, gs1, gs2, gs3)
    wsems = (ws0, ws1, ws2, ws3)

    def gather(ch, bslot):
        return pltpu.async_copy(table.at[idx_s.at[ch]], bufs[bslot],
                                gsems[bslot])

    def drain(sem, bslot):
        pltpu.make_async_copy(table.at[pl.ds(0, _G)], bufs[bslot], sem).wait()

    gather(0, 0)
    gather(1, 1)

    @pl.loop(0, _NCH, step=_NB)
    def _(i):
        for bslot in range(_NB):
            ch = i + bslot
            b = base_b + ch

            @pl.when(ch >= 2)
            def _():
                drain(wsems[(bslot + 2) % _NB], (bslot + 2) % _NB)

            @pl.when(ch + 2 < _NCH)
            def _():
                gather(ch + 2, (bslot + 2) % _NB)

            drain(gsems[bslot], bslot)
            pltpu.async_copy(bufs[bslot], outv.at[b], wsems[bslot])

    drain(wsems[(_NCH - 2) % _NB], (_NCH - 2) % _NB)
    drain(wsems[(_NCH - 1) % _NB], (_NCH - 1) % _NB)


@functools.lru_cache(maxsize=None)
def _vgather_call():
    return pl.kernel(
        _vgather_body,
        mesh=plsc.VectorSubcoreMesh(core_axis_name="c", subcore_axis_name="s"),
        out_type=jax.ShapeDtypeStruct((_B, _G, _HALF, _EMB_D), jnp.float32),
        scratch_types=(
            [pltpu.VMEM((_NCH, _G), jnp.int32)]
            + [pltpu.VMEM((_G, _HALF, _EMB_D), jnp.float32)] * _NB
            + [pltpu.SemaphoreType.DMA] * (2 * _NB)
        ),
    )


def kernel(query, p_0, k_0):
    idx, idxv, loss = _topk_call(query, k_0)
    table_v = p_0.reshape(_POOL * 2, _HALF, _EMB_D)
    idxv3 = idxv.reshape(_NW, _NCH, _G)
    outv = _vgather_call()(table_v, idxv3)
    p3 = p_0.reshape(_POOL * 2 * _SLAB, _L)
    outk = _kgather_call(p3, idx.reshape(_NROWS))
    prefix_k = outk.reshape(_B, _TOP_K * _HALF, _EMB_D)
    prefix_v = outv.reshape(_B, _TOP_K * _HALF, _EMB_D)
    return prefix_k, prefix_v, loss.reshape(())


# speedup vs baseline: 5.1853x; 5.1853x over previous
import functools

import jax
import jax.numpy as jnp
from jax import lax
from jax.experimental import pallas as pl
from jax.experimental.pallas import tpu as pltpu
from jax.experimental.pallas import tpu_sc as plsc

_EMB_D = 768
_KEY_D = 768
_POOL = 1024
_P_LEN = 8
_TOP_K = 8
_B = 1024
_HALF = _P_LEN // 2
_NROWS = _B * _TOP_K

_BB = 512
_NEG = jnp.finfo(jnp.float32).min

_NC = 2
_NS = 16
_NW = _NC * _NS
_G = _TOP_K
_NCH = _NROWS // _NW // _G


def _topk_body(q_ref, k_ref, idx_ref, loss_ref):
    i = pl.program_id(0)
    k = k_ref[...]
    kn = k / jnp.maximum(jnp.sqrt(jnp.sum(k * k, axis=1, keepdims=True)), 1e-12)
    q = q_ref[...]
    qn = q / jnp.maximum(jnp.sqrt(jnp.sum(q * q, axis=1, keepdims=True)), 1e-12)
    cos = lax.dot_general(qn, kn, (((1,), (1,)), ((), ())),
                          preferred_element_type=jnp.float32)
    col = lax.broadcasted_iota(jnp.int32, cos.shape, 1)
    s = cos
    idx_cols = []
    loss_acc = jnp.float32(0.0)
    for _ in range(_TOP_K):
        m = jnp.max(s, axis=1, keepdims=True)
        am = jnp.min(jnp.where(s == m, col, _POOL), axis=1,
                     keepdims=True)
        idx_cols.append(am)
        loss_acc = loss_acc + jnp.sum(1.0 - m)
        s = jnp.where(col == am, _NEG, s)
    idx_ref[...] = jnp.concatenate(idx_cols, axis=1)

    @pl.when(i == 0)
    def _():
        loss_ref[0, 0] = 0.0

    loss_ref[0, 0] += loss_acc / _B


def _topk_call(query, k_0):
    grid = _B // _BB
    return pl.pallas_call(
        _topk_body,
        grid=(grid,),
        in_specs=[
            pl.BlockSpec((_BB, _KEY_D), lambda i: (i, 0)),
            pl.BlockSpec((_POOL, _KEY_D), lambda i: (0, 0)),
        ],
        out_specs=[
            pl.BlockSpec((_BB, _TOP_K), lambda i: (i, 0)),
            pl.BlockSpec((1, 1), lambda i: (0, 0),
                         memory_space=pltpu.SMEM),
        ],
        out_shape=[
            jax.ShapeDtypeStruct((_B, _TOP_K), jnp.int32),
            jax.ShapeDtypeStruct((1, 1), jnp.float32),
        ],
    )(query, k_0)


def _gather_body(table, idx, outk, outv, idx_s, buf0, buf1,
                 gs0, gs1, ws0, ws1):
    c = lax.axis_index("c")
    s = lax.axis_index("s")
    wid = s * _NC + c
    base_b = wid * _NCH
    pltpu.sync_copy(idx.at[pl.ds(wid * _NCH, _NCH)], idx_s)

    bufs = (buf0, buf1)
    gsems = (gs0, gs1)
    wsems = (ws0, ws1)

    def gather(ch, bslot):
        return pltpu.async_copy(table.at[idx_s.at[ch]], bufs[bslot],
                                gsems[bslot])

    gh = [gather(0, 0), gather(1, 1)]

    @pl.loop(0, _NCH, step=2)
    def _(i):
        for bslot in range(2):
            ch = i + bslot
            b = base_b + ch
            gh[bslot].wait()
            hs = []
            for g in range(_G):
                hs.append(pltpu.async_copy(
                    bufs[bslot].at[g, pl.ds(0, _HALF), :],
                    outk.at[b, pl.ds(_HALF * g, _HALF), :], wsems[bslot]))
                hs.append(pltpu.async_copy(
                    bufs[bslot].at[g, pl.ds(_HALF, _HALF), :],
                    outv.at[b, pl.ds(_HALF * g, _HALF), :], wsems[bslot]))
            for h in hs:
                h.wait()

            @pl.when(ch + 2 < _NCH)
            def _():
                gather(ch + 2, bslot)


@functools.lru_cache(maxsize=None)
def _gather_call():
    return pl.kernel(
        _gather_body,
        mesh=plsc.VectorSubcoreMesh(core_axis_name="c", subcore_axis_name="s"),
        out_type=(
            jax.ShapeDtypeStruct((_B, _TOP_K * _HALF, _EMB_D), jnp.float32),
            jax.ShapeDtypeStruct((_B, _TOP_K * _HALF, _EMB_D), jnp.float32),
        ),
        scratch_types=(
            [pltpu.VMEM((_NCH, _G), jnp.int32)]
            + [pltpu.VMEM((_G, _P_LEN, _EMB_D), jnp.float32)] * 2
            + [pltpu.SemaphoreType.DMA] * 4
        ),
    )


def kernel(query, p_0, k_0):
    idx, loss = _topk_call(query, k_0)
    prefix_k, prefix_v = _gather_call()(p_0, idx)
    return prefix_k, prefix_v, loss.reshape(())
